# baseline (speedup 1.0000x reference)
import jax
import jax.numpy as jnp
from jax import lax
from jax.experimental import pallas as pl
from jax.experimental.pallas import tpu as pltpu
from jax.experimental.pallas import tpu_sc as plsc

B = 16384
C = 1000
MARGIN = 1.0
NC = 2
NS = 16
NW = NC * NS
COLS_W = B // NW
CG = 128
N_CG = COLS_W // CG
PR = 256
P_R0 = (0, 256, 512, 744)
P_LO = (0, 256, 512, 768)
P_HI = (256, 512, 768, 1000)
P_SKIP = (0, 0, 0, 24)

_sc_mesh = plsc.VectorSubcoreMesh(core_axis_name="c", subcore_axis_name="s")


def _hinge_sc(xt_hbm, y_hbm, out_hbm, y_v, xa, xb, outv, s1_v, g_sm,
              sem_y, sem_a, sem_b):
    wid = lax.axis_index("s") * NC + lax.axis_index("c")
    cbase = wid * COLS_W

    pltpu.make_async_copy(y_hbm.at[pl.ds(cbase, COLS_W)], y_v, sem_y).start()
    pltpu.make_async_copy(y_hbm.at[pl.ds(cbase, COLS_W)], y_v, sem_y).wait()

    bufs = (xa, xb)
    sems = (sem_a, sem_b)

    def dma(piece_idx, cg, p):
        c0 = cbase + cg * CG
        return pltpu.make_async_copy(
            xt_hbm.at[pl.ds(P_R0[p], PR), pl.ds(c0, CG)],
            bufs[piece_idx % 2], sems[piece_idx % 2])

    dma(0, 0, 0).start()
    dma(1, 0, 1).start()

    zeros = jnp.zeros((16,), jnp.float32)

    def body(cg, carry):
        tot, tot_s = carry
        for p in range(4):
            dma(p, cg, p).wait()
            buf = bufs[p % 2]

            def sgbody(sg, tot_in):
                def rbody(r, c):
                    s1, s2 = c
                    for k in range(8):
                        v = buf[P_SKIP[p] + r * 8 + k, pl.ds(sg * 16, 16)]
                        s1 = s1 + v
                        s2 = s2 + v * v
                    return s1, s2

                n_tr = (P_HI[p] - P_LO[p]) // 8
                s1p, s2p = lax.fori_loop(0, n_tr, rbody, (zeros, zeros))
                if p == 0:
                    s1_v[sg] = s1p
                else:
                    s1_v[sg] = s1_v[sg] + s1p
                y16 = y_v[pl.ds(cg * CG + sg * 16, 16)]
                for k in range(16):
                    yk = y16[k]
                    in_p = (yk >> 8) == p
                    row = jnp.clip(yk - P_R0[p], 0, PR - 1)
                    val = buf[row, pl.ds(sg * 16, 16)][k]
                    if p == 0:
                        g_sm[sg * 16 + k] = jnp.where(in_p, val, 0.0)
                    else:
                        g_sm[sg * 16 + k] = jnp.where(in_p, val,
                                                      g_sm[sg * 16 + k])
                return tot_in + s2p

            tot = lax.fori_loop(0, CG // 16, sgbody, tot)
            if p < 2:
                dma(p + 2, cg, p + 2).start()
            else:

                @pl.when(cg < N_CG - 1)
                def _():
                    dma(p + 2, cg + 1, p - 2).start()

        def algebra(sg, ts):
            s1vec = s1_v[sg]
            for k in range(16):
                g = g_sm[sg * 16 + k]
                omg = MARGIN - g
                ts = ts + 2.0 * omg * s1vec[k] + C * (omg * omg) - 1.0
            return ts

        tot_s = lax.fori_loop(0, CG // 16, algebra, tot_s)
        return tot, tot_s

    tot, tot_s = lax.fori_loop(0, N_CG, body, (zeros, jnp.float32(0.0)))
    outv[...] = tot + tot_s * 0.0625
    pltpu.make_async_copy(outv, out_hbm.at[wid], sem_y).start()
    pltpu.make_async_copy(outv, out_hbm.at[wid], sem_y).wait()


def kernel(output, y):
    xt = output.T
    partials = pl.kernel(
        _hinge_sc,
        mesh=_sc_mesh,
        out_type=jax.ShapeDtypeStruct((NW, 16), jnp.float32),
        scratch_types=[
            pltpu.VMEM((COLS_W,), jnp.int32),
            pltpu.VMEM((PR, CG), jnp.float32),
            pltpu.VMEM((PR, CG), jnp.float32),
            pltpu.VMEM((16,), jnp.float32),
            pltpu.VMEM((CG // 16, 16), jnp.float32),
            pltpu.SMEM((CG,), jnp.float32),
            pltpu.SemaphoreType.DMA,
            pltpu.SemaphoreType.DMA,
            pltpu.SemaphoreType.DMA,
        ],
    )(xt, y)
    return jnp.sum(partials) / B


# speedup vs baseline: 1.2293x; 1.2293x over previous
import jax
import jax.numpy as jnp
from jax import lax
from jax.experimental import pallas as pl
from jax.experimental.pallas import tpu as pltpu
from jax.experimental.pallas import tpu_sc as plsc

B = 16384
C = 1000
MARGIN = 1.0
NC = 2
NS = 16
NW = NC * NS
COLS_W = B // NW
CG = 128
N_CG = COLS_W // CG
PR = 256
P_R0 = (0, 256, 512, 744)
P_LO = (0, 256, 512, 768)
P_HI = (256, 512, 768, 1000)
P_SKIP = (0, 0, 0, 24)

_sc_mesh = plsc.VectorSubcoreMesh(core_axis_name="c", subcore_axis_name="s")


def _hinge_sc(xt_hbm, y_hbm, out_hbm, y_v, xa, xb, outv, s1_v, g_sm,
              sem_y, sem_a, sem_b):
    wid = lax.axis_index("s") * NC + lax.axis_index("c")
    cbase = wid * COLS_W

    pltpu.make_async_copy(y_hbm.at[pl.ds(cbase, COLS_W)], y_v, sem_y).start()
    pltpu.make_async_copy(y_hbm.at[pl.ds(cbase, COLS_W)], y_v, sem_y).wait()

    bufs = (xa, xb)
    sems = (sem_a, sem_b)

    def dma(piece_idx, cg, p):
        c0 = cbase + cg * CG
        return pltpu.make_async_copy(
            xt_hbm.at[pl.ds(P_R0[p], PR), pl.ds(c0, CG)],
            bufs[piece_idx % 2], sems[piece_idx % 2])

    dma(0, 0, 0).start()
    dma(1, 0, 1).start()

    zeros = jnp.zeros((16,), jnp.float32)

    def body(cg, carry):
        tot, tot_s = carry
        for p in range(4):
            dma(p, cg, p).wait()
            buf = bufs[p % 2]

            def sgbody(sg, tot_in):
                def rbody(r, c):
                    acc = list(c)
                    for k in range(8):
                        v = buf[P_SKIP[p] + r * 8 + k, pl.ds(sg * 16, 16)]
                        j = k % 4
                        acc[j] = acc[j] + v
                        acc[4 + j] = acc[4 + j] + v * v
                    return tuple(acc)

                n_tr = (P_HI[p] - P_LO[p]) // 8
                a = lax.fori_loop(0, n_tr, rbody, (zeros,) * 8)
                s1p = (a[0] + a[1]) + (a[2] + a[3])
                s2p = (a[4] + a[5]) + (a[6] + a[7])
                if p == 0:
                    s1_v[sg] = s1p
                else:
                    s1_v[sg] = s1_v[sg] + s1p
                y16 = y_v[pl.ds(cg * CG + sg * 16, 16)]
                for k in range(16):
                    yk = y16[k]
                    in_p = (yk >> 8) == p
                    row = jnp.clip(yk - P_R0[p], 0, PR - 1)
                    val = buf[row, pl.ds(sg * 16, 16)][k]
                    if p == 0:
                        g_sm[sg * 16 + k] = jnp.where(in_p, val, 0.0)
                    else:
                        g_sm[sg * 16 + k] = jnp.where(in_p, val,
                                                      g_sm[sg * 16 + k])
                return tot_in + s2p

            tot = lax.fori_loop(0, CG // 16, sgbody, tot)
            if p < 2:
                dma(p + 2, cg, p + 2).start()
            else:

                @pl.when(cg < N_CG - 1)
                def _():
                    dma(p + 2, cg + 1, p - 2).start()

        def algebra(sg, ts):
            s1vec = s1_v[sg]
            for k in range(16):
                g = g_sm[sg * 16 + k]
                omg = MARGIN - g
                ts = ts + 2.0 * omg * s1vec[k] + C * (omg * omg) - 1.0
            return ts

        tot_s = lax.fori_loop(0, CG // 16, algebra, tot_s)
        return tot, tot_s

    tot, tot_s = lax.fori_loop(0, N_CG, body, (zeros, jnp.float32(0.0)))
    outv[...] = tot + tot_s * 0.0625
    pltpu.make_async_copy(outv, out_hbm.at[wid], sem_y).start()
    pltpu.make_async_copy(outv, out_hbm.at[wid], sem_y).wait()


def kernel(output, y):
    xt = output.T
    partials = pl.kernel(
        _hinge_sc,
        mesh=_sc_mesh,
        out_type=jax.ShapeDtypeStruct((NW, 16), jnp.float32),
        scratch_types=[
            pltpu.VMEM((COLS_W,), jnp.int32),
            pltpu.VMEM((PR, CG), jnp.float32),
            pltpu.VMEM((PR, CG), jnp.float32),
            pltpu.VMEM((16,), jnp.float32),
            pltpu.VMEM((CG // 16, 16), jnp.float32),
            pltpu.SMEM((CG,), jnp.float32),
            pltpu.SemaphoreType.DMA,
            pltpu.SemaphoreType.DMA,
            pltpu.SemaphoreType.DMA,
        ],
    )(xt, y)
    return jnp.sum(partials) / B


# speedup vs baseline: 1.2305x; 1.0010x over previous
import jax
import jax.numpy as jnp
from jax import lax
from jax.experimental import pallas as pl
from jax.experimental.pallas import tpu as pltpu
from jax.experimental.pallas import tpu_sc as plsc

B = 16384
C = 1000
MARGIN = 1.0
NC = 2
NS = 16
NW = NC * NS
COLS_W = B // NW
CG = 128
N_CG = COLS_W // CG
PR = 256
P_R0 = (0, 256, 512, 744)
P_LO = (0, 256, 512, 768)
P_HI = (256, 512, 768, 1000)
P_SKIP = (0, 0, 0, 24)

_sc_mesh = plsc.VectorSubcoreMesh(core_axis_name="c", subcore_axis_name="s")


def _hinge_sc(xt_hbm, y_hbm, out_hbm, y_v, xa, xb, outv, s1_v, g_sm,
              sem_y, sem_a, sem_b):
    wid = lax.axis_index("s") * NC + lax.axis_index("c")
    cbase = wid * COLS_W

    pltpu.make_async_copy(y_hbm.at[pl.ds(cbase, COLS_W)], y_v, sem_y).start()
    pltpu.make_async_copy(y_hbm.at[pl.ds(cbase, COLS_W)], y_v, sem_y).wait()

    bufs = (xa, xb)
    sems = (sem_a, sem_b)

    def dma(piece_idx, cg, p):
        c0 = cbase + cg * CG
        return pltpu.make_async_copy(
            xt_hbm.at[pl.ds(P_R0[p], PR), pl.ds(c0, CG)],
            bufs[piece_idx % 2], sems[piece_idx % 2])

    dma(0, 0, 0).start()
    dma(1, 0, 1).start()

    zeros = jnp.zeros((16,), jnp.float32)

    def body(cg, carry):
        tot, tot_s = carry
        for p in range(4):
            dma(p, cg, p).wait()
            buf = bufs[p % 2]

            def sgbody(sg, tot_in):
                rpt = 16 if P_SKIP[p] == 0 else 8

                def rbody(r, c):
                    acc = list(c)
                    for k in range(rpt):
                        v = buf[P_SKIP[p] + r * rpt + k, pl.ds(sg * 16, 16)]
                        j = k % 4
                        acc[j] = acc[j] + v
                        acc[4 + j] = acc[4 + j] + v * v
                    return tuple(acc)

                n_tr = (P_HI[p] - P_LO[p]) // rpt
                a = lax.fori_loop(0, n_tr, rbody, (zeros,) * 8)
                s1p = (a[0] + a[1]) + (a[2] + a[3])
                s2p = (a[4] + a[5]) + (a[6] + a[7])
                if p == 0:
                    s1_v[sg] = s1p
                else:
                    s1_v[sg] = s1_v[sg] + s1p
                y16 = y_v[pl.ds(cg * CG + sg * 16, 16)]
                for k in range(16):
                    yk = y16[k]
                    in_p = (yk >> 8) == p
                    row = jnp.clip(yk - P_R0[p], 0, PR - 1)
                    val = buf[row, pl.ds(sg * 16, 16)][k]
                    if p == 0:
                        g_sm[sg * 16 + k] = jnp.where(in_p, val, 0.0)
                    else:
                        g_sm[sg * 16 + k] = jnp.where(in_p, val,
                                                      g_sm[sg * 16 + k])
                return tot_in + s2p

            tot = lax.fori_loop(0, CG // 16, sgbody, tot)
            if p < 2:
                dma(p + 2, cg, p + 2).start()
            else:

                @pl.when(cg < N_CG - 1)
                def _():
                    dma(p + 2, cg + 1, p - 2).start()

        def algebra(sg, ts):
            s1vec = s1_v[sg]
            for k in range(16):
                g = g_sm[sg * 16 + k]
                omg = MARGIN - g
                ts = ts + 2.0 * omg * s1vec[k] + C * (omg * omg) - 1.0
            return ts

        tot_s = lax.fori_loop(0, CG // 16, algebra, tot_s)
        return tot, tot_s

    tot, tot_s = lax.fori_loop(0, N_CG, body, (zeros, jnp.float32(0.0)))
    outv[...] = tot + tot_s * 0.0625
    pltpu.make_async_copy(outv, out_hbm.at[wid], sem_y).start()
    pltpu.make_async_copy(outv, out_hbm.at[wid], sem_y).wait()


def kernel(output, y):
    xt = output.T
    partials = pl.kernel(
        _hinge_sc,
        mesh=_sc_mesh,
        out_type=jax.ShapeDtypeStruct((NW, 16), jnp.float32),
        scratch_types=[
            pltpu.VMEM((COLS_W,), jnp.int32),
            pltpu.VMEM((PR, CG), jnp.float32),
            pltpu.VMEM((PR, CG), jnp.float32),
            pltpu.VMEM((16,), jnp.float32),
            pltpu.VMEM((CG // 16, 16), jnp.float32),
            pltpu.SMEM((CG,), jnp.float32),
            pltpu.SemaphoreType.DMA,
            pltpu.SemaphoreType.DMA,
            pltpu.SemaphoreType.DMA,
        ],
    )(xt, y)
    return jnp.sum(partials) / B


# speedup vs baseline: 2.2212x; 1.8050x over previous
import jax
import jax.numpy as jnp
from jax import lax
from jax.experimental import pallas as pl
from jax.experimental.pallas import tpu as pltpu
from jax.experimental.pallas import tpu_sc as plsc

B = 16384
C = 1000
MARGIN = 1.0

TC_COLS = 12288
BN = 2048
NB = TC_COLS // BN

NC = 2
NS = 16
NW = NC * NS
CG = 128
PR = 256
P_R0 = (0, 256, 512, 744)
P_LO = (0, 256, 512, 768)
P_HI = (256, 512, 768, 1000)
P_SKIP = (0, 0, 0, 24)

_sc_mesh = plsc.VectorSubcoreMesh(core_axis_name="c", subcore_axis_name="s")


def _hinge_tc(x_ref, y_ref, out_ref):
    i = pl.program_id(0)
    x = x_ref[...]
    yv = y_ref[...].reshape(1, BN)
    rows = jax.lax.broadcasted_iota(jnp.int32, (C, BN), 0)
    g = jnp.sum(jnp.where(rows == yv, x, 0.0), axis=0, keepdims=True)
    s1 = jnp.sum(x, axis=0, keepdims=True)
    s2 = jnp.sum(x * x, axis=0, keepdims=True)
    omg = MARGIN - g
    row_tot = s2 + 2.0 * omg * s1 + C * (omg * omg) - 1.0
    partial = jnp.sum(row_tot).reshape(1, 1)

    @pl.when(i == 0)
    def _init():
        out_ref[...] = jnp.zeros((1, 1), jnp.float32)

    out_ref[...] += partial


def _hinge_sc(xt_hbm, y_hbm, out_hbm, y_v, xa, xb, outv, s1_v, g_sm,
              sem_y, sem_a, sem_b):
    wid = lax.axis_index("s") * NC + lax.axis_index("c")
    cbase = TC_COLS + wid * CG

    pltpu.make_async_copy(y_hbm.at[pl.ds(cbase, CG)], y_v, sem_y).start()
    pltpu.make_async_copy(y_hbm.at[pl.ds(cbase, CG)], y_v, sem_y).wait()

    bufs = (xa, xb)
    sems = (sem_a, sem_b)

    def dma(p):
        return pltpu.make_async_copy(
            xt_hbm.at[pl.ds(P_R0[p], PR), pl.ds(cbase, CG)],
            bufs[p % 2], sems[p % 2])

    dma(0).start()
    dma(1).start()

    zeros = jnp.zeros((16,), jnp.float32)
    tot = zeros
    for p in range(4):
        dma(p).wait()
        buf = bufs[p % 2]

        def sgbody(sg, tot_in, p=p, buf=buf):
            rpt = 16 if P_SKIP[p] == 0 else 8

            def rbody(r, c):
                acc = list(c)
                for k in range(rpt):
                    v = buf[P_SKIP[p] + r * rpt + k, pl.ds(sg * 16, 16)]
                    j = k % 4
                    acc[j] = acc[j] + v
                    acc[4 + j] = acc[4 + j] + v * v
                return tuple(acc)

            n_tr = (P_HI[p] - P_LO[p]) // rpt
            a = lax.fori_loop(0, n_tr, rbody, (zeros,) * 8)
            s1p = (a[0] + a[1]) + (a[2] + a[3])
            s2p = (a[4] + a[5]) + (a[6] + a[7])
            if p == 0:
                s1_v[sg] = s1p
            else:
                s1_v[sg] = s1_v[sg] + s1p
            y16 = y_v[pl.ds(sg * 16, 16)]
            for k in range(16):
                yk = y16[k]
                in_p = (yk >> 8) == p
                row = jnp.clip(yk - P_R0[p], 0, PR - 1)
                val = buf[row, pl.ds(sg * 16, 16)][k]
                if p == 0:
                    g_sm[sg * 16 + k] = jnp.where(in_p, val, 0.0)
                else:
                    g_sm[sg * 16 + k] = jnp.where(in_p, val,
                                                  g_sm[sg * 16 + k])
            return tot_in + s2p

        tot = lax.fori_loop(0, CG // 16, sgbody, tot)
        if p < 2:
            dma(p + 2).start()

    def algebra(sg, ts):
        s1vec = s1_v[sg]
        for k in range(16):
            g = g_sm[sg * 16 + k]
            omg = MARGIN - g
            ts = ts + 2.0 * omg * s1vec[k] + C * (omg * omg) - 1.0
        return ts

    tot_s = lax.fori_loop(0, CG // 16, algebra, jnp.float32(0.0))
    outv[...] = tot + tot_s * 0.0625
    pltpu.make_async_copy(outv, out_hbm.at[wid], sem_y).start()
    pltpu.make_async_copy(outv, out_hbm.at[wid], sem_y).wait()


def kernel(output, y):
    xt = output.T

    sc_partials = pl.kernel(
        _hinge_sc,
        mesh=_sc_mesh,
        out_type=jax.ShapeDtypeStruct((NW, 16), jnp.float32),
        scratch_types=[
            pltpu.VMEM((CG,), jnp.int32),
            pltpu.VMEM((PR, CG), jnp.float32),
            pltpu.VMEM((PR, CG), jnp.float32),
            pltpu.VMEM((16,), jnp.float32),
            pltpu.VMEM((CG // 16, 16), jnp.float32),
            pltpu.SMEM((CG,), jnp.float32),
            pltpu.SemaphoreType.DMA,
            pltpu.SemaphoreType.DMA,
            pltpu.SemaphoreType.DMA,
        ],
    )(xt, y)

    tc_total = pl.pallas_call(
        _hinge_tc,
        grid=(NB,),
        in_specs=[
            pl.BlockSpec((C, BN), lambda i: (0, i)),
            pl.BlockSpec((BN,), lambda i: (i,)),
        ],
        out_specs=pl.BlockSpec((1, 1), lambda i: (0, 0)),
        out_shape=jax.ShapeDtypeStruct((1, 1), jnp.float32),
        compiler_params=pltpu.CompilerParams(
            dimension_semantics=("arbitrary",),
        ),
    )(xt, y)

    return (tc_total[0, 0] + jnp.sum(sc_partials)) / B


# speedup vs baseline: 4.0173x; 1.8086x over previous
import jax
import jax.numpy as jnp
from jax.experimental import pallas as pl
from jax.experimental.pallas import tpu as pltpu

B = 16384
C = 1000
MARGIN = 1.0
BN = 4096
NB = B // BN


def _hinge_block(x_ref, y_ref, out_ref):
    i = pl.program_id(0)
    x = x_ref[...]
    yv = y_ref[...].reshape(1, BN)
    rows = jax.lax.broadcasted_iota(jnp.int32, (C, BN), 0)
    g = jnp.sum(jnp.where(rows == yv, x, 0.0), axis=0, keepdims=True)
    s1 = jnp.sum(x, axis=0, keepdims=True)
    s2 = jnp.sum(x * x, axis=0, keepdims=True)
    omg = 1.0 - g
    row_tot = s2 + 2.0 * omg * s1 + C * (omg * omg) - 1.0
    partial = jnp.sum(row_tot).reshape(1, 1)

    @pl.when(i == 0)
    def _init():
        out_ref[...] = jnp.zeros((1, 1), jnp.float32)

    out_ref[...] += partial

    @pl.when(i == NB - 1)
    def _finish():
        out_ref[...] = out_ref[...] / B


def kernel(output, y):
    xt = output.T
    total = pl.pallas_call(
        _hinge_block,
        grid=(NB,),
        in_specs=[
            pl.BlockSpec((C, BN), lambda i: (0, i)),
            pl.BlockSpec((BN,), lambda i: (i,)),
        ],
        out_specs=pl.BlockSpec((1, 1), lambda i: (0, 0)),
        out_shape=jax.ShapeDtypeStruct((1, 1), jnp.float32),
        compiler_params=pltpu.CompilerParams(
            dimension_semantics=("arbitrary",),
        ),
    )(xt, y)
    return total[0, 0]


# speedup vs baseline: 4.0348x; 1.0044x over previous
import jax
import jax.numpy as jnp
from jax.experimental import pallas as pl
from jax.experimental.pallas import tpu as pltpu

B = 16384
C = 1000
MARGIN = 1.0
BN = 2048
NB = B // BN


def _hinge_block(x_ref, y_ref, out_ref):
    i = pl.program_id(0)
    x = x_ref[...]
    yv = y_ref[...].reshape(1, BN)
    rows = jax.lax.broadcasted_iota(jnp.int32, (C, BN), 0)
    g = jnp.sum(jnp.where(rows == yv, x, 0.0), axis=0, keepdims=True)
    s1 = jnp.sum(x, axis=0, keepdims=True)
    s2 = jnp.sum(x * x, axis=0, keepdims=True)
    omg = 1.0 - g
    row_tot = s2 + 2.0 * omg * s1 + C * (omg * omg) - 1.0
    partial = jnp.sum(row_tot).reshape(1, 1)

    @pl.when(i == 0)
    def _init():
        out_ref[...] = jnp.zeros((1, 1), jnp.float32)

    out_ref[...] += partial

    @pl.when(i == NB - 1)
    def _finish():
        out_ref[...] = out_ref[...] / B


def kernel(output, y):
    xt = output.T
    total = pl.pallas_call(
        _hinge_block,
        grid=(NB,),
        in_specs=[
            pl.BlockSpec((C, BN), lambda i: (0, i)),
            pl.BlockSpec((BN,), lambda i: (i,)),
        ],
        out_specs=pl.BlockSpec((1, 1), lambda i: (0, 0)),
        out_shape=jax.ShapeDtypeStruct((1, 1), jnp.float32),
        compiler_params=pltpu.CompilerParams(
            dimension_semantics=("arbitrary",),
        ),
    )(xt, y)
    return total[0, 0]
